# baseline (speedup 1.0000x reference)
import functools

import jax
import jax.numpy as jnp
from jax import lax
from jax.experimental import pallas as pl
from jax.experimental.pallas import tpu as pltpu
from jax.experimental.pallas import tpu_sc as plsc

UC = 100000
D = 128
L = 32
K = 16
B = 1024
VOCAB = 50000

NC, NS = 2, 16
NW = NC * NS
BPW = B // NW
GQ = BPW
GE = BPW * K
NBAG = 2 * GQ + 3 * GE


def _fold_body(w2v_ref, wl_ref, t_ref):
    t_ref[...] = jnp.dot(w2v_ref[...], wl_ref[...],
                         preferred_element_type=jnp.float32) * (1.0 / L)


def _fold_table(w2v, w_lstm):
    blk = 512
    grid = (VOCAB + blk - 1) // blk
    return pl.pallas_call(
        _fold_body,
        grid=(grid,),
        in_specs=[
            pl.BlockSpec((blk, D), lambda i: (i, 0)),
            pl.BlockSpec((D, D), lambda i: (0, 0)),
        ],
        out_specs=pl.BlockSpec((blk, D), lambda i: (i, 0)),
        out_shape=jax.ShapeDtypeStruct((VOCAB, D), jnp.float32),
    )(w2v, w_lstm)


def _reduce_bag(rows, outbuf, lrow):
    for c in range(D // 16):
        acc = rows[0, pl.ds(c * 16, 16)]
        for r in range(1, L):
            acc = acc + rows[r, pl.ds(c * 16, 16)]
        outbuf[lrow, pl.ds(c * 16, 16)] = acc


def _bag_group(t_hbm, cont, rows0, rows1, outbuf, sem0, sem1, off, nbags,
               out_hbm, obase):
    last = off + nbags - 1
    pltpu.async_copy(t_hbm.at[cont.at[off]], rows0, sem0)
    pltpu.async_copy(t_hbm.at[cont.at[off + 1]], rows1, sem1)

    def pair(p, carry):
        bag = 2 * p
        lrow = bag - (bag // BPW) * BPW

        pltpu.make_async_copy(t_hbm.at[pl.ds(0, L)], rows0, sem0).wait()
        _reduce_bag(rows0, outbuf, lrow)
        nxt0 = jnp.minimum(off + bag + 2, last)
        pltpu.async_copy(t_hbm.at[cont.at[nxt0]], rows0, sem0)

        pltpu.make_async_copy(t_hbm.at[pl.ds(0, L)], rows1, sem1).wait()
        _reduce_bag(rows1, outbuf, lrow + 1)
        nxt1 = jnp.minimum(off + bag + 3, last)
        pltpu.async_copy(t_hbm.at[cont.at[nxt1]], rows1, sem1)

        @pl.when(lrow == BPW - 2)
        def _():
            pltpu.sync_copy(outbuf,
                            out_hbm.at[pl.ds(obase + bag - (BPW - 2), BPW)])

        return carry

    lax.fori_loop(0, nbags // 2, pair, 0)
    pltpu.make_async_copy(t_hbm.at[pl.ds(0, L)], rows0, sem0).wait()
    pltpu.make_async_copy(t_hbm.at[pl.ds(0, L)], rows1, sem1).wait()


CPIECE = 64


def _sc_body(q_hbm, ae_hbm, u_hbm, adj_hbm, adje_hbm, cont_hbm, t_hbm, ut_hbm,
             qh_o, ah_o, qeh_o, unh_o, ueh_o, us_o, qnb_o,
             qv, uv, av, adjq, aeq, adju, aeu, ids_v, cont, rows0, rows1,
             outbuf, qidx, sem0, sem1):
    w = lax.axis_index("s") * NC + lax.axis_index("c")
    b0 = w * BPW

    pltpu.sync_copy(q_hbm.at[pl.ds(b0, BPW)], qv)
    pltpu.sync_copy(u_hbm.at[pl.ds(b0, BPW)], uv)
    pltpu.sync_copy(ae_hbm.at[pl.ds(b0, BPW)], av)

    pltpu.async_copy(adj_hbm.at[qv], adjq, sem0).wait()
    pltpu.async_copy(adje_hbm.at[qv], aeq, sem0).wait()
    pltpu.async_copy(adj_hbm.at[uv], adju, sem0).wait()
    pltpu.async_copy(adje_hbm.at[uv], aeu, sem0).wait()

    for i in range(BPW // 16):
        ids_v[pl.ds(i * 16, 16)] = qv[pl.ds(i * 16, 16)] - UC
        ids_v[pl.ds(GQ + i * 16, 16)] = av[pl.ds(i * 16, 16)] - UC
    for r in range(BPW):
        ids_v[pl.ds(2 * GQ + r * K, K)] = aeq[r, :] - UC
        ids_v[pl.ds(2 * GQ + GE + r * K, K)] = adju[r, :] - UC
        ids_v[pl.ds(2 * GQ + 2 * GE + r * K, K)] = aeu[r, :] - UC
        qidx[pl.ds(r * K, K)] = adjq[r, :]

    def cont_piece(p, carry):
        pltpu.async_copy(
            cont_hbm.at[ids_v.at[pl.ds(p * CPIECE, CPIECE)]],
            cont.at[pl.ds(p * CPIECE, CPIECE)], sem0).wait()
        return carry

    lax.fori_loop(0, NBAG // CPIECE, cont_piece, 0)

    _bag_group(t_hbm, cont, rows0, rows1, outbuf, sem0, sem1, 0, GQ, qh_o, b0)
    _bag_group(t_hbm, cont, rows0, rows1, outbuf, sem0, sem1, GQ, GQ, ah_o, b0)
    _bag_group(t_hbm, cont, rows0, rows1, outbuf, sem0, sem1, 2 * GQ, GE,
               qeh_o, b0 * K)
    _bag_group(t_hbm, cont, rows0, rows1, outbuf, sem0, sem1, 2 * GQ + GE, GE,
               unh_o, b0 * K)
    _bag_group(t_hbm, cont, rows0, rows1, outbuf, sem0, sem1,
               2 * GQ + 2 * GE, GE, ueh_o, b0 * K)

    pltpu.async_copy(ut_hbm.at[uv], rows0, sem0).wait()
    pltpu.sync_copy(rows0, us_o.at[pl.ds(b0, BPW)])

    for j in range(GE // BPW):
        pltpu.async_copy(ut_hbm.at[qidx.at[pl.ds(j * BPW, BPW)]], rows0,
                         sem0).wait()
        pltpu.sync_copy(rows0, qnb_o.at[pl.ds(b0 * K + j * BPW, BPW)])


def _sc_gather(question, answer_edge, user, adj, adj_edge, content, t,
               user_table):
    out_type = (
        jax.ShapeDtypeStruct((B, D), jnp.float32),
        jax.ShapeDtypeStruct((B, D), jnp.float32),
        jax.ShapeDtypeStruct((B * K, D), jnp.float32),
        jax.ShapeDtypeStruct((B * K, D), jnp.float32),
        jax.ShapeDtypeStruct((B * K, D), jnp.float32),
        jax.ShapeDtypeStruct((B, D), jnp.float32),
        jax.ShapeDtypeStruct((B * K, D), jnp.float32),
    )
    scratch_types = [
        pltpu.VMEM((BPW,), jnp.int32),
        pltpu.VMEM((BPW,), jnp.int32),
        pltpu.VMEM((BPW,), jnp.int32),
        pltpu.VMEM((BPW, K), jnp.int32),
        pltpu.VMEM((BPW, K), jnp.int32),
        pltpu.VMEM((BPW, K), jnp.int32),
        pltpu.VMEM((BPW, K), jnp.int32),
        pltpu.VMEM((NBAG,), jnp.int32),
        pltpu.VMEM((NBAG, L), jnp.int32),
        pltpu.VMEM((L, D), jnp.float32),
        pltpu.VMEM((L, D), jnp.float32),
        pltpu.VMEM((BPW, D), jnp.float32),
        pltpu.VMEM((BPW * K,), jnp.int32),
        pltpu.SemaphoreType.DMA,
        pltpu.SemaphoreType.DMA,
    ]
    fn = pl.kernel(
        _sc_body,
        out_type=out_type,
        scratch_types=scratch_types,
        compiler_params=pltpu.CompilerParams(use_tc_tiling_on_sc=False),
        mesh=plsc.VectorSubcoreMesh(core_axis_name="c", subcore_axis_name="s"),
    )
    return fn(question, answer_edge, user, adj, adj_edge, content, t,
              user_table)


BB = 128


def _dense_body(qh, ah, qeh, unh, ueh, us, qnb,
                blstm, wn_u, we_u, b_u, wn_q, we_q, b_q,
                wg_su, wg_nu, wg_sq, wg_nq,
                wq, bq, wa, ba, wu, bu, wf, bf,
                logp_o, pred_o):
    bl = blstm[...]
    q_self = jnp.tanh(qh[...] + bl)
    a_emb = jnp.tanh(ah[...] + bl)
    qe_m = jnp.mean(jnp.tanh(qeh[...] + bl).reshape(BB, K, D), axis=1)
    un_m = jnp.mean(jnp.tanh(unh[...] + bl).reshape(BB, K, D), axis=1)
    ue_m = jnp.mean(jnp.tanh(ueh[...] + bl).reshape(BB, K, D), axis=1)
    qnb_m = jnp.mean(qnb[...].reshape(BB, K, D), axis=1)
    u_self = us[...]

    def mm(x, wref):
        return jnp.dot(x, wref[...], preferred_element_type=jnp.float32)

    qedge_m = qe_m + 0.5 * (qnb_m + q_self)
    q_agg = jax.nn.relu(mm(qnb_m, wn_u) + mm(qedge_m, we_u) + b_u[...])
    q0 = jax.nn.relu(mm(q_self, wg_su) + mm(q_agg, wg_nu))

    uedge_m = ue_m + 0.5 * (un_m + u_self)
    u_agg = jax.nn.relu(mm(un_m, wn_q) + mm(uedge_m, we_q) + b_q[...])
    u0 = jax.nn.relu(mm(u_self, wg_sq) + mm(u_agg, wg_nq))

    score = jnp.tanh(mm(a_emb, wa) + ba[...] + mm(q0, wq) + bq[...]
                     + mm(u0, wu) + bu[...])
    logits = mm(score, wf) + bf[...]
    m = jnp.max(logits, axis=-1, keepdims=True)
    lse = m + jnp.log(jnp.sum(jnp.exp(logits - m), axis=-1, keepdims=True))
    logp_o[...] = logits - lse
    pred_o[...] = (logits[:, 1:2] > logits[:, 0:1]).astype(jnp.int32)


def _dense(qh, ah, qeh, unh, ueh, us, qnb, p):
    grid = (B // BB,)
    row = lambda i: (i, 0)
    fix = lambda i: (0, 0)
    bspec = lambda shape, im: pl.BlockSpec(shape, im)
    in_specs = [
        bspec((BB, D), row), bspec((BB, D), row),
        bspec((BB * K, D), row), bspec((BB * K, D), row),
        bspec((BB * K, D), row),
        bspec((BB, D), row), bspec((BB * K, D), row),
        bspec((1, D), fix),
        bspec((D, D), fix), bspec((D, D), fix), bspec((1, D), fix),
        bspec((D, D), fix), bspec((D, D), fix), bspec((1, D), fix),
        bspec((D, D), fix), bspec((D, D), fix),
        bspec((D, D), fix), bspec((D, D), fix),
        bspec((D, D), fix), bspec((1, D), fix),
        bspec((D, D), fix), bspec((1, D), fix),
        bspec((D, D), fix), bspec((1, D), fix),
        bspec((D, 2), fix), bspec((1, 2), fix),
    ]
    r2 = lambda a: a.reshape(1, -1)
    out = pl.pallas_call(
        _dense_body,
        grid=grid,
        in_specs=in_specs,
        out_specs=[pl.BlockSpec((BB, 2), row), pl.BlockSpec((BB, 1), row)],
        out_shape=[
            jax.ShapeDtypeStruct((B, 2), jnp.float32),
            jax.ShapeDtypeStruct((B, 1), jnp.int32),
        ],
    )(qh, ah, qeh, unh, ueh, us, qnb,
      r2(p['b_lstm']),
      p['Wn_u'], p['We_u'], r2(p['b_u_agg']),
      p['Wn_q'], p['We_q'], r2(p['b_q_agg']),
      p['Wg_self_u'], p['Wg_nb_u'], p['Wg_self_q'], p['Wg_nb_q'],
      p['Wq'], r2(p['bq']), p['Wa'], r2(p['ba']), p['Wu'], r2(p['bu']),
      p['Wf'], r2(p['bf']))
    return out[0], out[1][:, 0]


def kernel(question, answer_edge, user, adj, adj_edge, content, params):
    t = _fold_table(params['word2vec'], params['W_lstm'])
    qh, ah, qeh, unh, ueh, us, qnb = _sc_gather(
        question, answer_edge, user, adj, adj_edge, content, t,
        params['user_table'])
    return _dense(qh, ah, qeh, unh, ueh, us, qnb, params)


# speedup vs baseline: 1.2291x; 1.2291x over previous
import functools

import jax
import jax.numpy as jnp
from jax import lax
from jax.experimental import pallas as pl
from jax.experimental.pallas import tpu as pltpu
from jax.experimental.pallas import tpu_sc as plsc

UC = 100000
D = 128
L = 32
K = 16
B = 1024
VOCAB = 50000

NC, NS = 2, 16
NW = NC * NS
BPW = B // NW
GQ = BPW
GE = BPW * K
NBAG = 2 * GQ + 3 * GE


def _fold_body(w2v_ref, wl_ref, t_ref):
    t_ref[...] = jnp.dot(w2v_ref[...], wl_ref[...],
                         preferred_element_type=jnp.float32) * (1.0 / L)


def _fold_table(w2v, w_lstm):
    blk = 512
    grid = (VOCAB + blk - 1) // blk
    return pl.pallas_call(
        _fold_body,
        grid=(grid,),
        in_specs=[
            pl.BlockSpec((blk, D), lambda i: (i, 0)),
            pl.BlockSpec((D, D), lambda i: (0, 0)),
        ],
        out_specs=pl.BlockSpec((blk, D), lambda i: (i, 0)),
        out_shape=jax.ShapeDtypeStruct((VOCAB, D), jnp.float32),
    )(w2v, w_lstm)


CB = 4
NBUF = 3
CW = CB * L


def _reduce_chunk(rows, outbuf, lrow):
    def one(b, carry):
        rb = b * L
        for c in range(D // 16):
            acc = rows[rb, pl.ds(c * 16, 16)]
            for r in range(1, L):
                acc = acc + rows[rb + r, pl.ds(c * 16, 16)]
            outbuf[lrow + b, pl.ds(c * 16, 16)] = acc
        return carry

    lax.fori_loop(0, CB, one, 0)


def _bag_all(t_hbm, contf, bufs, sems, outbuf, flush):
    nchunks = NBAG // CB
    lastc = nchunks - 1

    def fire(i, chunk):
        pltpu.async_copy(t_hbm.at[contf.at[pl.ds(chunk * CW, CW)]], bufs[i],
                         sems[i])

    def drain(i):
        pltpu.make_async_copy(t_hbm.at[contf.at[pl.ds(0, CW)]], bufs[i],
                              sems[i]).wait()

    def consume(i, bagbase):
        lrow = bagbase - (bagbase // BPW) * BPW
        drain(i)
        _reduce_chunk(bufs[i], outbuf, lrow)

        @pl.when(lrow == BPW - CB)
        def _():
            flush(bagbase - (BPW - CB))

    for i in range(NBUF):
        fire(i, i)

    def step(s, carry):
        chunk = NBUF * s
        for i in range(NBUF):
            consume(i, (chunk + i) * CB)
            fire(i, jnp.minimum(chunk + i + NBUF, lastc))
        return carry

    lax.fori_loop(0, nchunks // NBUF, step, 0)
    rem = nchunks - (nchunks // NBUF) * NBUF
    for i in range(NBUF):
        if i < rem:
            consume(i, (nchunks - rem + i) * CB)
        else:
            drain(i)


CPIECE = 64


def _sc_body(q_hbm, ae_hbm, u_hbm, adj_hbm, adje_hbm, cont_hbm, t_hbm, ut_hbm,
             qh_o, ah_o, qeh_o, unh_o, ueh_o, us_o, qnb_o,
             qv, uv, av, adjq, aeq, adju, aeu, ids_v, contf, pieceb,
             buf0, buf1, buf2, outbuf, qidx, sem0, sem1, sem2):
    bufs = (buf0, buf1, buf2)
    sems = (sem0, sem1, sem2)
    w = lax.axis_index("s") * NC + lax.axis_index("c")
    b0 = w * BPW

    pltpu.sync_copy(q_hbm.at[pl.ds(b0, BPW)], qv)
    pltpu.sync_copy(u_hbm.at[pl.ds(b0, BPW)], uv)
    pltpu.sync_copy(ae_hbm.at[pl.ds(b0, BPW)], av)

    pltpu.async_copy(adj_hbm.at[qv], adjq, sem0).wait()
    pltpu.async_copy(adje_hbm.at[qv], aeq, sem0).wait()
    pltpu.async_copy(adj_hbm.at[uv], adju, sem0).wait()
    pltpu.async_copy(adje_hbm.at[uv], aeu, sem0).wait()

    for i in range(BPW // 16):
        ids_v[pl.ds(i * 16, 16)] = qv[pl.ds(i * 16, 16)] - UC
        ids_v[pl.ds(GQ + i * 16, 16)] = av[pl.ds(i * 16, 16)] - UC
    for r in range(BPW):
        ids_v[pl.ds(2 * GQ + r * K, K)] = aeq[r, :] - UC
        ids_v[pl.ds(2 * GQ + GE + r * K, K)] = adju[r, :] - UC
        ids_v[pl.ds(2 * GQ + 2 * GE + r * K, K)] = aeu[r, :] - UC
        qidx[pl.ds(r * K, K)] = adjq[r, :]

    def cont_piece(p, carry):
        pltpu.async_copy(
            cont_hbm.at[ids_v.at[pl.ds(p * CPIECE, CPIECE)]],
            pieceb, sem0).wait()
        base = p * CPIECE * L
        for g in range(CPIECE * L // 16):
            contf[pl.ds(base + g * 16, 16)] = pieceb[g // 2,
                                                     pl.ds((g % 2) * 16, 16)]
        return carry

    lax.fori_loop(0, NBAG // CPIECE, cont_piece, 0)

    def flush(gstart):
        @pl.when(gstart < GQ)
        def _():
            pltpu.sync_copy(outbuf, qh_o.at[pl.ds(b0, BPW)])

        @pl.when(jnp.logical_and(gstart >= GQ, gstart < 2 * GQ))
        def _():
            pltpu.sync_copy(outbuf, ah_o.at[pl.ds(b0, BPW)])

        @pl.when(jnp.logical_and(gstart >= 2 * GQ, gstart < 2 * GQ + GE))
        def _():
            pltpu.sync_copy(
                outbuf, qeh_o.at[pl.ds(b0 * K + gstart - 2 * GQ, BPW)])

        @pl.when(jnp.logical_and(gstart >= 2 * GQ + GE,
                                 gstart < 2 * GQ + 2 * GE))
        def _():
            pltpu.sync_copy(
                outbuf, unh_o.at[pl.ds(b0 * K + gstart - (2 * GQ + GE), BPW)])

        @pl.when(gstart >= 2 * GQ + 2 * GE)
        def _():
            pltpu.sync_copy(
                outbuf,
                ueh_o.at[pl.ds(b0 * K + gstart - (2 * GQ + 2 * GE), BPW)])

    _bag_all(t_hbm, contf, bufs, sems, outbuf, flush)

    rows0 = buf0.at[pl.ds(0, BPW)]
    pltpu.async_copy(ut_hbm.at[uv], rows0, sem0).wait()
    pltpu.sync_copy(rows0, us_o.at[pl.ds(b0, BPW)])

    for j in range(GE // BPW):
        pltpu.async_copy(ut_hbm.at[qidx.at[pl.ds(j * BPW, BPW)]], rows0,
                         sem0).wait()
        pltpu.sync_copy(rows0, qnb_o.at[pl.ds(b0 * K + j * BPW, BPW)])


def _sc_gather(question, answer_edge, user, adj, adj_edge, content, t,
               user_table):
    out_type = (
        jax.ShapeDtypeStruct((B, D), jnp.float32),
        jax.ShapeDtypeStruct((B, D), jnp.float32),
        jax.ShapeDtypeStruct((B * K, D), jnp.float32),
        jax.ShapeDtypeStruct((B * K, D), jnp.float32),
        jax.ShapeDtypeStruct((B * K, D), jnp.float32),
        jax.ShapeDtypeStruct((B, D), jnp.float32),
        jax.ShapeDtypeStruct((B * K, D), jnp.float32),
    )
    scratch_types = [
        pltpu.VMEM((BPW,), jnp.int32),
        pltpu.VMEM((BPW,), jnp.int32),
        pltpu.VMEM((BPW,), jnp.int32),
        pltpu.VMEM((BPW, K), jnp.int32),
        pltpu.VMEM((BPW, K), jnp.int32),
        pltpu.VMEM((BPW, K), jnp.int32),
        pltpu.VMEM((BPW, K), jnp.int32),
        pltpu.VMEM((NBAG,), jnp.int32),
        pltpu.VMEM((NBAG * L,), jnp.int32),
        pltpu.VMEM((CPIECE, L), jnp.int32),
        pltpu.VMEM((CB * L, D), jnp.float32),
        pltpu.VMEM((CB * L, D), jnp.float32),
        pltpu.VMEM((CB * L, D), jnp.float32),
        pltpu.VMEM((BPW, D), jnp.float32),
        pltpu.VMEM((BPW * K,), jnp.int32),
        pltpu.SemaphoreType.DMA,
        pltpu.SemaphoreType.DMA,
        pltpu.SemaphoreType.DMA,
    ]
    fn = pl.kernel(
        _sc_body,
        out_type=out_type,
        scratch_types=scratch_types,
        compiler_params=pltpu.CompilerParams(use_tc_tiling_on_sc=False),
        mesh=plsc.VectorSubcoreMesh(core_axis_name="c", subcore_axis_name="s"),
    )
    return fn(question, answer_edge, user, adj, adj_edge, content, t,
              user_table)


BB = 128


def _dense_body(qh, ah, qeh, unh, ueh, us, qnb,
                blstm, wn_u, we_u, b_u, wn_q, we_q, b_q,
                wg_su, wg_nu, wg_sq, wg_nq,
                wq, bq, wa, ba, wu, bu, wf, bf,
                logp_o, pred_o):
    bl = blstm[...]
    q_self = jnp.tanh(qh[...] + bl)
    a_emb = jnp.tanh(ah[...] + bl)
    qe_m = jnp.mean(jnp.tanh(qeh[...] + bl).reshape(BB, K, D), axis=1)
    un_m = jnp.mean(jnp.tanh(unh[...] + bl).reshape(BB, K, D), axis=1)
    ue_m = jnp.mean(jnp.tanh(ueh[...] + bl).reshape(BB, K, D), axis=1)
    qnb_m = jnp.mean(qnb[...].reshape(BB, K, D), axis=1)
    u_self = us[...]

    def mm(x, wref):
        return jnp.dot(x, wref[...], preferred_element_type=jnp.float32)

    qedge_m = qe_m + 0.5 * (qnb_m + q_self)
    q_agg = jax.nn.relu(mm(qnb_m, wn_u) + mm(qedge_m, we_u) + b_u[...])
    q0 = jax.nn.relu(mm(q_self, wg_su) + mm(q_agg, wg_nu))

    uedge_m = ue_m + 0.5 * (un_m + u_self)
    u_agg = jax.nn.relu(mm(un_m, wn_q) + mm(uedge_m, we_q) + b_q[...])
    u0 = jax.nn.relu(mm(u_self, wg_sq) + mm(u_agg, wg_nq))

    score = jnp.tanh(mm(a_emb, wa) + ba[...] + mm(q0, wq) + bq[...]
                     + mm(u0, wu) + bu[...])
    logits = mm(score, wf) + bf[...]
    m = jnp.max(logits, axis=-1, keepdims=True)
    lse = m + jnp.log(jnp.sum(jnp.exp(logits - m), axis=-1, keepdims=True))
    logp_o[...] = logits - lse
    pred_o[...] = (logits[:, 1:2] > logits[:, 0:1]).astype(jnp.int32)


def _dense(qh, ah, qeh, unh, ueh, us, qnb, p):
    grid = (B // BB,)
    row = lambda i: (i, 0)
    fix = lambda i: (0, 0)
    bspec = lambda shape, im: pl.BlockSpec(shape, im)
    in_specs = [
        bspec((BB, D), row), bspec((BB, D), row),
        bspec((BB * K, D), row), bspec((BB * K, D), row),
        bspec((BB * K, D), row),
        bspec((BB, D), row), bspec((BB * K, D), row),
        bspec((1, D), fix),
        bspec((D, D), fix), bspec((D, D), fix), bspec((1, D), fix),
        bspec((D, D), fix), bspec((D, D), fix), bspec((1, D), fix),
        bspec((D, D), fix), bspec((D, D), fix),
        bspec((D, D), fix), bspec((D, D), fix),
        bspec((D, D), fix), bspec((1, D), fix),
        bspec((D, D), fix), bspec((1, D), fix),
        bspec((D, D), fix), bspec((1, D), fix),
        bspec((D, 2), fix), bspec((1, 2), fix),
    ]
    r2 = lambda a: a.reshape(1, -1)
    out = pl.pallas_call(
        _dense_body,
        grid=grid,
        in_specs=in_specs,
        out_specs=[pl.BlockSpec((BB, 2), row), pl.BlockSpec((BB, 1), row)],
        out_shape=[
            jax.ShapeDtypeStruct((B, 2), jnp.float32),
            jax.ShapeDtypeStruct((B, 1), jnp.int32),
        ],
    )(qh, ah, qeh, unh, ueh, us, qnb,
      r2(p['b_lstm']),
      p['Wn_u'], p['We_u'], r2(p['b_u_agg']),
      p['Wn_q'], p['We_q'], r2(p['b_q_agg']),
      p['Wg_self_u'], p['Wg_nb_u'], p['Wg_self_q'], p['Wg_nb_q'],
      p['Wq'], r2(p['bq']), p['Wa'], r2(p['ba']), p['Wu'], r2(p['bu']),
      p['Wf'], r2(p['bf']))
    return out[0], out[1][:, 0]


def kernel(question, answer_edge, user, adj, adj_edge, content, params):
    t = _fold_table(params['word2vec'], params['W_lstm'])
    qh, ah, qeh, unh, ueh, us, qnb = _sc_gather(
        question, answer_edge, user, adj, adj_edge, content, t,
        params['user_table'])
    return _dense(qh, ah, qeh, unh, ueh, us, qnb, params)


# speedup vs baseline: 1.2602x; 1.0253x over previous
import functools

import jax
import jax.numpy as jnp
from jax import lax
from jax.experimental import pallas as pl
from jax.experimental.pallas import tpu as pltpu
from jax.experimental.pallas import tpu_sc as plsc

UC = 100000
D = 128
L = 32
K = 16
B = 1024
VOCAB = 50000

NC, NS = 2, 16
NW = NC * NS
BPW = B // NW
GQ = BPW
GE = BPW * K
NBAG = 2 * GQ + 3 * GE


def _fold_body(w2v_ref, wl_ref, t_ref):
    t_ref[...] = jnp.dot(w2v_ref[...], wl_ref[...],
                         preferred_element_type=jnp.float32) * (1.0 / L)


def _fold_table(w2v, w_lstm):
    blk = 512
    grid = (VOCAB + blk - 1) // blk
    return pl.pallas_call(
        _fold_body,
        grid=(grid,),
        in_specs=[
            pl.BlockSpec((blk, D), lambda i: (i, 0)),
            pl.BlockSpec((D, D), lambda i: (0, 0)),
        ],
        out_specs=pl.BlockSpec((blk, D), lambda i: (i, 0)),
        out_shape=jax.ShapeDtypeStruct((VOCAB, D), jnp.float32),
    )(w2v, w_lstm)


CB = 4
NBUF = 4
CW = CB * L
HALF = NBAG // 2


def _reduce_chunk(rows, outbuf, lrow):
    def one(b, carry):
        rb = b * L
        for c in range(D // 16):
            acc = rows[rb, pl.ds(c * 16, 16)]
            for r in range(1, L):
                acc = acc + rows[rb + r, pl.ds(c * 16, 16)]
            outbuf[lrow + b, pl.ds(c * 16, 16)] = acc
        return carry

    lax.fori_loop(0, CB, one, 0)


def _bag_half(t_hbm, contf, bufs, sems, outbuf, flush, half_off):
    nchunks = HALF // CB
    lastc = nchunks - 1

    def fire(i, chunk):
        pltpu.async_copy(t_hbm.at[contf.at[pl.ds(chunk * CW, CW)]], bufs[i],
                         sems[i])

    def drain(i):
        pltpu.make_async_copy(t_hbm.at[contf.at[pl.ds(0, CW)]], bufs[i],
                              sems[i]).wait()

    def consume(i, bagbase):
        lrow = bagbase - (bagbase // BPW) * BPW
        drain(i)
        _reduce_chunk(bufs[i], outbuf, lrow)

        @pl.when(lrow == BPW - CB)
        def _():
            flush(half_off + bagbase - (BPW - CB))

    for i in range(NBUF):
        fire(i, i)

    def step(s, carry):
        chunk = NBUF * s
        for i in range(NBUF):
            consume(i, (chunk + i) * CB)
            fire(i, jnp.minimum(chunk + i + NBUF, lastc))
        return carry

    lax.fori_loop(0, nchunks // NBUF, step, 0)
    for i in range(NBUF):
        drain(i)


CPIECE = 80


def _sc_body(q_hbm, ae_hbm, u_hbm, adj_hbm, adje_hbm, cont_hbm, t_hbm, ut_hbm,
             qh_o, ah_o, qeh_o, unh_o, ueh_o, us_o, qnb_o,
             qv, uv, av, adjq, aeq, adju, aeu, ids_v, contf, pieceb,
             buf0, buf1, buf2, buf3, outbuf, qidx, sem0, sem1, sem2, sem3):
    bufs = (buf0, buf1, buf2, buf3)
    sems = (sem0, sem1, sem2, sem3)
    w = lax.axis_index("s") * NC + lax.axis_index("c")
    b0 = w * BPW

    pltpu.sync_copy(q_hbm.at[pl.ds(b0, BPW)], qv)
    pltpu.sync_copy(u_hbm.at[pl.ds(b0, BPW)], uv)
    pltpu.sync_copy(ae_hbm.at[pl.ds(b0, BPW)], av)

    pltpu.async_copy(adj_hbm.at[qv], adjq, sem0).wait()
    pltpu.async_copy(adje_hbm.at[qv], aeq, sem0).wait()
    pltpu.async_copy(adj_hbm.at[uv], adju, sem0).wait()
    pltpu.async_copy(adje_hbm.at[uv], aeu, sem0).wait()

    for i in range(BPW // 16):
        ids_v[pl.ds(i * 16, 16)] = qv[pl.ds(i * 16, 16)] - UC
        ids_v[pl.ds(GQ + i * 16, 16)] = av[pl.ds(i * 16, 16)] - UC
    for r in range(BPW):
        ids_v[pl.ds(2 * GQ + r * K, K)] = aeq[r, :] - UC
        ids_v[pl.ds(2 * GQ + GE + r * K, K)] = adju[r, :] - UC
        ids_v[pl.ds(2 * GQ + 2 * GE + r * K, K)] = aeu[r, :] - UC
        qidx[pl.ds(r * K, K)] = adjq[r, :]

    def cont_half(half_off):
        def cont_piece(p, carry):
            pltpu.async_copy(
                cont_hbm.at[ids_v.at[pl.ds(half_off + p * CPIECE, CPIECE)]],
                pieceb, sem0).wait()
            base = p * CPIECE * L
            for g in range(CPIECE * L // 16):
                contf[pl.ds(base + g * 16, 16)] = pieceb[
                    g // 2, pl.ds((g % 2) * 16, 16)]
            return carry

        lax.fori_loop(0, HALF // CPIECE, cont_piece, 0)

    def flush(gstart):
        @pl.when(gstart < GQ)
        def _():
            pltpu.sync_copy(outbuf, qh_o.at[pl.ds(b0, BPW)])

        @pl.when(jnp.logical_and(gstart >= GQ, gstart < 2 * GQ))
        def _():
            pltpu.sync_copy(outbuf, ah_o.at[pl.ds(b0, BPW)])

        @pl.when(jnp.logical_and(gstart >= 2 * GQ, gstart < 2 * GQ + GE))
        def _():
            pltpu.sync_copy(
                outbuf, qeh_o.at[pl.ds(b0 * K + gstart - 2 * GQ, BPW)])

        @pl.when(jnp.logical_and(gstart >= 2 * GQ + GE,
                                 gstart < 2 * GQ + 2 * GE))
        def _():
            pltpu.sync_copy(
                outbuf, unh_o.at[pl.ds(b0 * K + gstart - (2 * GQ + GE), BPW)])

        @pl.when(gstart >= 2 * GQ + 2 * GE)
        def _():
            pltpu.sync_copy(
                outbuf,
                ueh_o.at[pl.ds(b0 * K + gstart - (2 * GQ + 2 * GE), BPW)])

    for h in range(2):
        cont_half(h * HALF)
        _bag_half(t_hbm, contf, bufs, sems, outbuf, flush, h * HALF)

    rows0 = buf0.at[pl.ds(0, BPW)]
    pltpu.async_copy(ut_hbm.at[uv], rows0, sem0).wait()
    pltpu.sync_copy(rows0, us_o.at[pl.ds(b0, BPW)])

    for j in range(GE // BPW):
        pltpu.async_copy(ut_hbm.at[qidx.at[pl.ds(j * BPW, BPW)]], rows0,
                         sem0).wait()
        pltpu.sync_copy(rows0, qnb_o.at[pl.ds(b0 * K + j * BPW, BPW)])


def _sc_gather(question, answer_edge, user, adj, adj_edge, content, t,
               user_table):
    out_type = (
        jax.ShapeDtypeStruct((B, D), jnp.float32),
        jax.ShapeDtypeStruct((B, D), jnp.float32),
        jax.ShapeDtypeStruct((B * K, D), jnp.float32),
        jax.ShapeDtypeStruct((B * K, D), jnp.float32),
        jax.ShapeDtypeStruct((B * K, D), jnp.float32),
        jax.ShapeDtypeStruct((B, D), jnp.float32),
        jax.ShapeDtypeStruct((B * K, D), jnp.float32),
    )
    scratch_types = [
        pltpu.VMEM((BPW,), jnp.int32),
        pltpu.VMEM((BPW,), jnp.int32),
        pltpu.VMEM((BPW,), jnp.int32),
        pltpu.VMEM((BPW, K), jnp.int32),
        pltpu.VMEM((BPW, K), jnp.int32),
        pltpu.VMEM((BPW, K), jnp.int32),
        pltpu.VMEM((BPW, K), jnp.int32),
        pltpu.VMEM((NBAG,), jnp.int32),
        pltpu.VMEM((HALF * L,), jnp.int32),
        pltpu.VMEM((CPIECE, L), jnp.int32),
        pltpu.VMEM((CB * L, D), jnp.float32),
        pltpu.VMEM((CB * L, D), jnp.float32),
        pltpu.VMEM((CB * L, D), jnp.float32),
        pltpu.VMEM((CB * L, D), jnp.float32),
        pltpu.VMEM((BPW, D), jnp.float32),
        pltpu.VMEM((BPW * K,), jnp.int32),
        pltpu.SemaphoreType.DMA,
        pltpu.SemaphoreType.DMA,
        pltpu.SemaphoreType.DMA,
        pltpu.SemaphoreType.DMA,
    ]
    fn = pl.kernel(
        _sc_body,
        out_type=out_type,
        scratch_types=scratch_types,
        compiler_params=pltpu.CompilerParams(use_tc_tiling_on_sc=False),
        mesh=plsc.VectorSubcoreMesh(core_axis_name="c", subcore_axis_name="s"),
    )
    return fn(question, answer_edge, user, adj, adj_edge, content, t,
              user_table)


BB = 128


def _dense_body(qh, ah, qeh, unh, ueh, us, qnb,
                blstm, wn_u, we_u, b_u, wn_q, we_q, b_q,
                wg_su, wg_nu, wg_sq, wg_nq,
                wq, bq, wa, ba, wu, bu, wf, bf,
                logp_o, pred_o):
    bl = blstm[...]
    q_self = jnp.tanh(qh[...] + bl)
    a_emb = jnp.tanh(ah[...] + bl)
    qe_m = jnp.mean(jnp.tanh(qeh[...] + bl).reshape(BB, K, D), axis=1)
    un_m = jnp.mean(jnp.tanh(unh[...] + bl).reshape(BB, K, D), axis=1)
    ue_m = jnp.mean(jnp.tanh(ueh[...] + bl).reshape(BB, K, D), axis=1)
    qnb_m = jnp.mean(qnb[...].reshape(BB, K, D), axis=1)
    u_self = us[...]

    def mm(x, wref):
        return jnp.dot(x, wref[...], preferred_element_type=jnp.float32)

    qedge_m = qe_m + 0.5 * (qnb_m + q_self)
    q_agg = jax.nn.relu(mm(qnb_m, wn_u) + mm(qedge_m, we_u) + b_u[...])
    q0 = jax.nn.relu(mm(q_self, wg_su) + mm(q_agg, wg_nu))

    uedge_m = ue_m + 0.5 * (un_m + u_self)
    u_agg = jax.nn.relu(mm(un_m, wn_q) + mm(uedge_m, we_q) + b_q[...])
    u0 = jax.nn.relu(mm(u_self, wg_sq) + mm(u_agg, wg_nq))

    score = jnp.tanh(mm(a_emb, wa) + ba[...] + mm(q0, wq) + bq[...]
                     + mm(u0, wu) + bu[...])
    logits = mm(score, wf) + bf[...]
    m = jnp.max(logits, axis=-1, keepdims=True)
    lse = m + jnp.log(jnp.sum(jnp.exp(logits - m), axis=-1, keepdims=True))
    logp_o[...] = logits - lse
    pred_o[...] = (logits[:, 1:2] > logits[:, 0:1]).astype(jnp.int32)


def _dense(qh, ah, qeh, unh, ueh, us, qnb, p):
    grid = (B // BB,)
    row = lambda i: (i, 0)
    fix = lambda i: (0, 0)
    bspec = lambda shape, im: pl.BlockSpec(shape, im)
    in_specs = [
        bspec((BB, D), row), bspec((BB, D), row),
        bspec((BB * K, D), row), bspec((BB * K, D), row),
        bspec((BB * K, D), row),
        bspec((BB, D), row), bspec((BB * K, D), row),
        bspec((1, D), fix),
        bspec((D, D), fix), bspec((D, D), fix), bspec((1, D), fix),
        bspec((D, D), fix), bspec((D, D), fix), bspec((1, D), fix),
        bspec((D, D), fix), bspec((D, D), fix),
        bspec((D, D), fix), bspec((D, D), fix),
        bspec((D, D), fix), bspec((1, D), fix),
        bspec((D, D), fix), bspec((1, D), fix),
        bspec((D, D), fix), bspec((1, D), fix),
        bspec((D, 2), fix), bspec((1, 2), fix),
    ]
    r2 = lambda a: a.reshape(1, -1)
    out = pl.pallas_call(
        _dense_body,
        grid=grid,
        in_specs=in_specs,
        out_specs=[pl.BlockSpec((BB, 2), row), pl.BlockSpec((BB, 1), row)],
        out_shape=[
            jax.ShapeDtypeStruct((B, 2), jnp.float32),
            jax.ShapeDtypeStruct((B, 1), jnp.int32),
        ],
    )(qh, ah, qeh, unh, ueh, us, qnb,
      r2(p['b_lstm']),
      p['Wn_u'], p['We_u'], r2(p['b_u_agg']),
      p['Wn_q'], p['We_q'], r2(p['b_q_agg']),
      p['Wg_self_u'], p['Wg_nb_u'], p['Wg_self_q'], p['Wg_nb_q'],
      p['Wq'], r2(p['bq']), p['Wa'], r2(p['ba']), p['Wu'], r2(p['bu']),
      p['Wf'], r2(p['bf']))
    return out[0], out[1][:, 0]


def kernel(question, answer_edge, user, adj, adj_edge, content, params):
    t = _fold_table(params['word2vec'], params['W_lstm'])
    qh, ah, qeh, unh, ueh, us, qnb = _sc_gather(
        question, answer_edge, user, adj, adj_edge, content, t,
        params['user_table'])
    return _dense(qh, ah, qeh, unh, ueh, us, qnb, params)
